# baseline (speedup 1.0000x reference)
import jax
import jax.numpy as jnp
from jax import lax
from jax.experimental import pallas as pl
from jax.experimental.pallas import tpu as pltpu
from jax.experimental.pallas import tpu_sc as plsc

BATCH = 4096
HIST = 50
EMBED = 128
TOTAL = BATCH * HIST
NC = 2
NS = 16
NW = NC * NS
BAT_PER_W = BATCH // NW
B_PER_W = BAT_PER_W * HIST
CHB = 8
CH = CHB * HIST
N_CHUNKS = BAT_PER_W // CHB
NBUF = 2
LOOK = 1
N_OUTER = N_CHUNKS // NBUF


def _gather_body(idx_hbm, table_hbm, out_hbm, idx_v,
                 rows0, rows1,
                 gsem0, gsem1,
                 ssem0, ssem1):
    wid = lax.axis_index("s") * NC + lax.axis_index("c")
    wbase = wid * B_PER_W
    wbat = wid * BAT_PER_W
    rows = (rows0, rows1)
    gsem = (gsem0, gsem1)
    ssem = (ssem0, ssem1)

    pltpu.sync_copy(idx_hbm.at[pl.ds(wbase, B_PER_W)], idx_v)

    def gather_start(j, b):
        pltpu.async_copy(
            table_hbm.at[idx_v.at[pl.ds(j * CH, CH)]], rows[b], gsem[b])

    def process(j, b):
        pltpu.make_async_copy(
            table_hbm.at[idx_v.at[pl.ds(0, CH)]], rows[b], gsem[b]).wait()
        for k in range(CHB):
            pltpu.async_copy(
                rows[b].at[pl.ds(k * HIST, HIST)],
                out_hbm.at[wbat + j * CHB + k], ssem[b])

    def drain_store(b):
        for _ in range(CHB):
            pltpu.make_async_copy(
                rows[b].at[pl.ds(0, HIST)], out_hbm.at[wbat], ssem[b]).wait()

    for b in range(LOOK):
        gather_start(b, b)

    for b in range(NBUF):
        process(b, b)
        jn = b + LOOK
        bn = jn % NBUF
        if jn >= NBUF:
            drain_store(bn)
        gather_start(jn, bn)

    def body(g, carry):
        j0 = g * NBUF
        for b in range(NBUF):
            j = j0 + b
            process(j, b)
            bn = (b + LOOK) % NBUF

            @pl.when(j + LOOK < N_CHUNKS)
            def _():
                drain_store(bn)
                gather_start(j + LOOK, bn)
        return carry

    lax.fori_loop(1, N_OUTER, body, 0)

    for b in range(NBUF):
        drain_store(b)


def kernel(indices, table):
    flat_idx = indices.reshape(TOTAL)
    mesh = plsc.VectorSubcoreMesh(core_axis_name="c", subcore_axis_name="s")
    k = pl.kernel(
        _gather_body,
        mesh=mesh,
        out_type=jax.ShapeDtypeStruct((BATCH, HIST, EMBED), jnp.float32),
        scratch_types=(
            [pltpu.VMEM((B_PER_W,), jnp.int32)]
            + [pltpu.VMEM((CH, EMBED), jnp.float32)] * NBUF
            + [pltpu.SemaphoreType.DMA] * (2 * NBUF)
        ),
    )
    return k(flat_idx, table)


# speedup vs baseline: 1.0142x; 1.0142x over previous
import jax
import jax.numpy as jnp
from jax import lax
from jax.experimental import pallas as pl
from jax.experimental.pallas import tpu as pltpu
from jax.experimental.pallas import tpu_sc as plsc

BATCH = 4096
HIST = 50
EMBED = 128
TOTAL = BATCH * HIST
NC = 2
NS = 16
NW = NC * NS
BAT_PER_W = BATCH // NW
B_PER_W = BAT_PER_W * HIST
CHB = 4
CH = CHB * HIST
N_CHUNKS = BAT_PER_W // CHB
NBUF = 4
LOOK = 3
N_OUTER = N_CHUNKS // NBUF


def _gather_body(idx_hbm, table_hbm, out_hbm, idx_v,
                 rows0, rows1, rows2, rows3,
                 gsem0, gsem1, gsem2, gsem3,
                 ssem0, ssem1, ssem2, ssem3):
    wid = lax.axis_index("s") * NC + lax.axis_index("c")
    wbase = wid * B_PER_W
    wbat = wid * BAT_PER_W
    rows = (rows0, rows1, rows2, rows3)
    gsem = (gsem0, gsem1, gsem2, gsem3)
    ssem = (ssem0, ssem1, ssem2, ssem3)

    pltpu.sync_copy(idx_hbm.at[pl.ds(wbase, B_PER_W)], idx_v)

    def gather_start(j, b):
        pltpu.async_copy(
            table_hbm.at[idx_v.at[pl.ds(j * CH, CH)]], rows[b], gsem[b])

    def process(j, b):
        pltpu.make_async_copy(
            table_hbm.at[idx_v.at[pl.ds(0, CH)]], rows[b], gsem[b]).wait()
        for k in range(CHB):
            pltpu.async_copy(
                rows[b].at[pl.ds(k * HIST, HIST)],
                out_hbm.at[wbat + j * CHB + k], ssem[b])

    def drain_store(b):
        for _ in range(CHB):
            pltpu.make_async_copy(
                rows[b].at[pl.ds(0, HIST)], out_hbm.at[wbat], ssem[b]).wait()

    for b in range(LOOK):
        gather_start(b, b)

    for b in range(NBUF):
        process(b, b)
        jn = b + LOOK
        bn = jn % NBUF
        if jn >= NBUF:
            drain_store(bn)
        gather_start(jn, bn)

    def body(g, carry):
        j0 = g * NBUF
        for b in range(NBUF):
            j = j0 + b
            process(j, b)
            bn = (b + LOOK) % NBUF

            @pl.when(j + LOOK < N_CHUNKS)
            def _():
                drain_store(bn)
                gather_start(j + LOOK, bn)
        return carry

    lax.fori_loop(1, N_OUTER, body, 0)

    for b in range(NBUF):
        drain_store(b)


def kernel(indices, table):
    flat_idx = indices.reshape(TOTAL)
    mesh = plsc.VectorSubcoreMesh(core_axis_name="c", subcore_axis_name="s")
    k = pl.kernel(
        _gather_body,
        mesh=mesh,
        out_type=jax.ShapeDtypeStruct((BATCH, HIST, EMBED), jnp.float32),
        scratch_types=(
            [pltpu.VMEM((B_PER_W,), jnp.int32)]
            + [pltpu.VMEM((CH, EMBED), jnp.float32)] * NBUF
            + [pltpu.SemaphoreType.DMA] * (2 * NBUF)
        ),
    )
    return k(flat_idx, table)
